# baseline (speedup 1.0000x reference)
import functools

import jax
import jax.numpy as jnp
from jax import lax
from jax.experimental import pallas as pl
from jax.experimental.pallas import tpu as pltpu
from jax.experimental.pallas import tpu_sc as plsc

N = 10000
NPAD = 10240
E = 320000
EPAD = 327680
INC = 128
HID = 256
CH = 128
NG = 64
RB = 256
NBLK = NPAD // RB
K = 128
EPC = EPAD // 16
EPD = EPAD // 32
DEGR = NPAD // 128
RPT = NPAD // 16

_mesh = plsc.VectorSubcoreMesh(core_axis_name="c", subcore_axis_name="s")


NDCH = EPD // K
DG = 8


@functools.partial(
    pl.kernel,
    out_type=jax.ShapeDtypeStruct((2 * NPAD, CH), jnp.float32),
    mesh=_mesh,
    scratch_types=[
        pltpu.VMEM((NDCH, K), jnp.int32),
        pltpu.VMEM((K, CH), jnp.float32),
        pltpu.VMEM_SHARED((NPAD, CH), jnp.float32),
        pltpu.SemaphoreType.DMA,
        pltpu.SemaphoreType.DMA,
    ],
)
def _deg_kernel(dst3_hbm, ones_hbm, zz_hbm, out_hbm,
                dsts_v, ones_v, deg_sh, semA, semB):
    c = lax.axis_index("c")
    s = lax.axis_index("s")
    wid = s * 2 + c
    pltpu.sync_copy(ones_hbm, ones_v)
    pltpu.sync_copy(dst3_hbm.at[wid], dsts_v)
    pltpu.sync_copy(
        zz_hbm.at[pl.ds(s * RPT, RPT)], deg_sh.at[pl.ds(s * RPT, RPT)]
    )
    plsc.subcore_barrier()

    def fire(g, sem):
        for t in range(DG):
            pltpu.async_copy(
                ones_v, deg_sh.at[dsts_v.at[g * DG + t]], sem, add=True
            )

    def drain(g, sem):
        for t in range(DG):
            pltpu.make_async_copy(
                ones_v, deg_sh.at[dsts_v.at[g * DG + t]], sem
            ).wait()

    NGRP = NDCH // DG
    fire(0, semA)
    for g in range(NGRP):
        cur = semA if g % 2 == 0 else semB
        nxt = semB if g % 2 == 0 else semA
        if g + 1 < NGRP:
            fire(g + 1, nxt)
        drain(g, cur)

    plsc.subcore_barrier()
    pltpu.sync_copy(
        deg_sh.at[pl.ds(s * RPT, RPT)],
        out_hbm.at[pl.ds(c * NPAD + s * RPT, RPT)],
    )


NCH = EPC // K
IB = 20
NBLKS = NCH // IB


@functools.partial(
    pl.kernel,
    out_type=jax.ShapeDtypeStruct((2 * NPAD, CH), jnp.float32),
    mesh=_mesh,
    scratch_types=[
        pltpu.VMEM((IB, 2, K), jnp.int32),
        pltpu.VMEM((IB, 2, K), jnp.int32),
        pltpu.VMEM((K, CH), jnp.float32),
        pltpu.VMEM((K, CH), jnp.float32),
        pltpu.VMEM_SHARED((NPAD, CH), jnp.float32),
        pltpu.SemaphoreType.DMA,
        pltpu.SemaphoreType.DMA,
        pltpu.SemaphoreType.DMA,
        pltpu.SemaphoreType.DMA,
    ],
)
def _agg_kernel(s_hbm, sd4_hbm, zz_hbm, out_hbm,
                sd0, sd1, rows0, rows1, acc_sh, semx0, semx1, semg0, semg1):
    c = lax.axis_index("c")
    s = lax.axis_index("s")
    w = c * 16 + s
    pltpu.async_copy(sd4_hbm.at[w, pl.ds(0, IB)], sd0, semx0)
    pltpu.sync_copy(
        zz_hbm.at[pl.ds(s * RPT, RPT)], acc_sh.at[pl.ds(s * RPT, RPT)]
    )
    plsc.subcore_barrier()

    for b in range(NBLKS):
        sd_c, semx_c = (sd0, semx0) if b % 2 == 0 else (sd1, semx1)
        sd_n, semx_n = (sd1, semx1) if b % 2 == 0 else (sd0, semx0)
        if b + 1 < NBLKS:
            pltpu.async_copy(
                sd4_hbm.at[w, pl.ds((b + 1) * IB, IB)], sd_n, semx_n
            )
        pltpu.make_async_copy(
            sd4_hbm.at[w, pl.ds(b * IB, IB)], sd_c, semx_c
        ).wait()

        pltpu.async_copy(s_hbm.at[sd_c.at[0, 0]], rows0, semg0)

        def body(i, carry, sd_c=sd_c):
            j0 = i * 2
            j1 = j0 + 1
            pltpu.async_copy(s_hbm.at[sd_c.at[j1, 0]], rows1, semg1)
            pltpu.make_async_copy(s_hbm.at[sd_c.at[j0, 0]], rows0, semg0).wait()
            pltpu.sync_copy(rows0, acc_sh.at[sd_c.at[j0, 1]], add=True)

            @pl.when(i < IB // 2 - 1)
            def _():
                pltpu.async_copy(s_hbm.at[sd_c.at[j0 + 2, 0]], rows0, semg0)

            pltpu.make_async_copy(s_hbm.at[sd_c.at[j1, 0]], rows1, semg1).wait()
            pltpu.sync_copy(rows1, acc_sh.at[sd_c.at[j1, 1]], add=True)
            return carry

        lax.fori_loop(0, IB // 2, body, 0)

    plsc.subcore_barrier()
    pltpu.sync_copy(
        acc_sh.at[pl.ds(s * RPT, RPT)],
        out_hbm.at[pl.ds(c * NPAD + s * RPT, RPT)],
    )


def _b1a_body(x_ref, w_ref, o_ref):
    h = jnp.dot(x_ref[...], w_ref[...], preferred_element_type=jnp.float32)
    o_ref[0] = h[:, :CH]
    o_ref[1] = h[:, CH:]


def _b1b_body(r_ref, g0_ref, g1_ref, s_ref, d_ref):
    d = lax.rsqrt(g0_ref[0, 0] + g1_ref[0, 0] + 1.0)
    s_ref[0] = r_ref[0] * d[:, None]
    s_ref[1] = r_ref[1] * d[:, None]
    d_ref[0, 0] = d


def _b2_body(a_ref, s_ref, d_ref, b_ref, w_ref, o_ref):
    i = pl.program_id(0)
    a = jnp.concatenate([a_ref[0], a_ref[1]], axis=1)
    sf = jnp.concatenate([s_ref[0], s_ref[1]], axis=1)
    d = d_ref[0, 0]
    h = jnp.maximum((a + sf) * d[:, None] + b_ref[0][None, :], 0.0)
    s2 = jnp.dot(h, w_ref[...], preferred_element_type=jnp.float32)
    s2 = s2 * d[:, None]
    rid = i * RB + lax.broadcasted_iota(jnp.int32, (RB, 1), 0)
    s2 = jnp.where(rid < N, s2, 0.0)
    o_ref[0] = s2[:, :CH]
    o_ref[1] = s2[:, CH:]


def _b3_body(a_ref, s_ref, d_ref, b_ref, bat_ref, lw_ref, os_ref, oc_ref):
    i = pl.program_id(0)
    a = jnp.concatenate([a_ref[0], a_ref[1]], axis=1)
    sf = jnp.concatenate([s_ref[0], s_ref[1]], axis=1)
    d = d_ref[0, 0]
    h = jnp.maximum((a + sf) * d[:, None] + b_ref[0][None, :], 0.0)
    y = jnp.dot(h, lw_ref[...], preferred_element_type=jnp.float32)
    bat = bat_ref[0, 0]
    gid = lax.broadcasted_iota(jnp.int32, (NG, 1), 0)
    m = (bat[None, :] == gid).astype(jnp.float32)
    ps = jnp.dot(m, y, preferred_element_type=jnp.float32)
    pc = jnp.broadcast_to(jnp.sum(m, axis=1, keepdims=True), (NG, 128))

    @pl.when(i == 0)
    def _():
        os_ref[...] = jnp.zeros_like(os_ref)
        oc_ref[...] = jnp.zeros_like(oc_ref)

    os_ref[...] += ps
    oc_ref[...] += pc


def _b1a_call(xp, W1):
    return pl.pallas_call(
        _b1a_body,
        grid=(NBLK,),
        in_specs=[
            pl.BlockSpec((RB, INC), lambda i: (i, 0)),
            pl.BlockSpec((INC, HID), lambda i: (0, 0)),
        ],
        out_specs=pl.BlockSpec((2, RB, CH), lambda i: (0, i, 0)),
        out_shape=jax.ShapeDtypeStruct((2, NPAD, CH), jnp.float32),
    )(xp, W1)


def _b1b_call(raw, deg0, deg1):
    return pl.pallas_call(
        _b1b_body,
        grid=(NBLK,),
        in_specs=[
            pl.BlockSpec((2, RB, CH), lambda i: (0, i, 0)),
            pl.BlockSpec((1, 1, RB), lambda i: (i, 0, 0)),
            pl.BlockSpec((1, 1, RB), lambda i: (i, 0, 0)),
        ],
        out_specs=[
            pl.BlockSpec((2, RB, CH), lambda i: (0, i, 0)),
            pl.BlockSpec((1, 1, RB), lambda i: (i, 0, 0)),
        ],
        out_shape=[
            jax.ShapeDtypeStruct((2, NPAD, CH), jnp.float32),
            jax.ShapeDtypeStruct((NBLK, 1, RB), jnp.float32),
        ],
    )(raw, deg0, deg1)


def _b2_call(agg1, s1, d, b1r, W2):
    return pl.pallas_call(
        _b2_body,
        grid=(NBLK,),
        in_specs=[
            pl.BlockSpec((2, RB, CH), lambda i: (0, i, 0)),
            pl.BlockSpec((2, RB, CH), lambda i: (0, i, 0)),
            pl.BlockSpec((1, 1, RB), lambda i: (i, 0, 0)),
            pl.BlockSpec((1, HID), lambda i: (0, 0)),
            pl.BlockSpec((HID, HID), lambda i: (0, 0)),
        ],
        out_specs=pl.BlockSpec((2, RB, CH), lambda i: (0, i, 0)),
        out_shape=jax.ShapeDtypeStruct((2, NPAD, CH), jnp.float32),
    )(agg1, s1, d, b1r, W2)


def _b3_call(agg2, s2, d, b2r, batp, lwp):
    return pl.pallas_call(
        _b3_body,
        grid=(NBLK,),
        in_specs=[
            pl.BlockSpec((2, RB, CH), lambda i: (0, i, 0)),
            pl.BlockSpec((2, RB, CH), lambda i: (0, i, 0)),
            pl.BlockSpec((1, 1, RB), lambda i: (i, 0, 0)),
            pl.BlockSpec((1, HID), lambda i: (0, 0)),
            pl.BlockSpec((1, 1, RB), lambda i: (i, 0, 0)),
            pl.BlockSpec((HID, 128), lambda i: (0, 0)),
        ],
        out_specs=[
            pl.BlockSpec((NG, 128), lambda i: (0, 0)),
            pl.BlockSpec((NG, 128), lambda i: (0, 0)),
        ],
        out_shape=[
            jax.ShapeDtypeStruct((NG, 128), jnp.float32),
            jax.ShapeDtypeStruct((NG, 128), jnp.float32),
        ],
    )(agg2, s2, d, b2r, batp, lwp)


def kernel(x, edge_index, batch, W1, b1, W2, b2, lin_W, lin_b):
    src = edge_index[0].astype(jnp.int32)
    dst = edge_index[1].astype(jnp.int32)
    pad = jnp.full((EPAD - E,), N, jnp.int32)
    srcp = jnp.concatenate([src, pad])
    dstp = jnp.concatenate([dst, pad])
    src_r = srcp.reshape(16, EPC // K, 1, K)
    dst_r = dstp.reshape(16, EPC // K, 1, K)
    sd4 = jnp.concatenate(
        [
            jnp.concatenate([src_r, dst_r], axis=2),
            jnp.concatenate([src_r + NPAD, dst_r], axis=2),
        ],
        axis=0,
    )
    xp = jnp.zeros((NPAD, INC), jnp.float32).at[:N].set(x)
    zz = jnp.zeros((NPAD, CH), jnp.float32)
    batp = jnp.concatenate(
        [batch.astype(jnp.int32), jnp.full((NPAD - N,), NG, jnp.int32)]
    ).reshape(NBLK, 1, RB)
    b1r = b1.reshape(1, HID)
    b2r = b2.reshape(1, HID)
    lwp = jnp.pad(lin_W, ((0, 0), (0, 128 - lin_W.shape[1])))

    ones1 = jnp.zeros((K, CH), jnp.float32).at[:, 0].set(1.0)
    dst4 = dstp.reshape(32, EPD // K, K)
    raw1 = _b1a_call(xp, W1)
    degout = _deg_kernel(dst4, ones1, zz)
    deg0 = degout[:NPAD, 0].reshape(NBLK, 1, RB)
    deg1 = degout[NPAD:, 0].reshape(NBLK, 1, RB)

    s1, d = _b1b_call(raw1, deg0, deg1)
    agg1 = _agg_kernel(s1.reshape(2 * NPAD, CH), sd4, zz)
    s2 = _b2_call(agg1.reshape(2, NPAD, CH), s1, d, b1r, W2)
    agg2 = _agg_kernel(s2.reshape(2 * NPAD, CH), sd4, zz)
    osum, ocnt = _b3_call(agg2.reshape(2, NPAD, CH), s2, d, b2r, batp, lwp)

    return osum[:, 0] / jnp.maximum(ocnt[:, 0], 1.0) + lin_b[0]


# speedup vs baseline: 1.0629x; 1.0629x over previous
import functools

import jax
import jax.numpy as jnp
from jax import lax
from jax.experimental import pallas as pl
from jax.experimental.pallas import tpu as pltpu
from jax.experimental.pallas import tpu_sc as plsc

N = 10000
NPAD = 10240
E = 320000
EPAD = 327680
INC = 128
HID = 256
CH = 128
NG = 64
RB = 256
NBLK = NPAD // RB
K = 128
EPC = EPAD // 16
EPD = EPAD // 32
DEGR = NPAD // 128
RPT = NPAD // 16

_mesh = plsc.VectorSubcoreMesh(core_axis_name="c", subcore_axis_name="s")


NDCH = EPD // K
DG = 8


@functools.partial(
    pl.kernel,
    out_type=jax.ShapeDtypeStruct((2 * NPAD, CH), jnp.float32),
    mesh=_mesh,
    scratch_types=[
        pltpu.VMEM((NDCH, K), jnp.int32),
        pltpu.VMEM((K, CH), jnp.float32),
        pltpu.VMEM_SHARED((NPAD, CH), jnp.float32),
        pltpu.SemaphoreType.DMA,
        pltpu.SemaphoreType.DMA,
    ],
)
def _deg_kernel(dst3_hbm, ones_hbm, zz_hbm, out_hbm,
                dsts_v, ones_v, deg_sh, semA, semB):
    c = lax.axis_index("c")
    s = lax.axis_index("s")
    wid = s * 2 + c
    pltpu.sync_copy(ones_hbm, ones_v)
    pltpu.sync_copy(dst3_hbm.at[wid], dsts_v)
    pltpu.sync_copy(
        zz_hbm.at[pl.ds(s * RPT, RPT)], deg_sh.at[pl.ds(s * RPT, RPT)]
    )
    plsc.subcore_barrier()

    def fire(g, sem):
        for t in range(DG):
            pltpu.async_copy(
                ones_v, deg_sh.at[dsts_v.at[g * DG + t]], sem, add=True
            )

    def drain(g, sem):
        for t in range(DG):
            pltpu.make_async_copy(
                ones_v, deg_sh.at[dsts_v.at[g * DG + t]], sem
            ).wait()

    NGRP = NDCH // DG
    fire(0, semA)
    for g in range(NGRP):
        cur = semA if g % 2 == 0 else semB
        nxt = semB if g % 2 == 0 else semA
        if g + 1 < NGRP:
            fire(g + 1, nxt)
        drain(g, cur)

    plsc.subcore_barrier()
    pltpu.sync_copy(
        deg_sh.at[pl.ds(s * RPT, RPT)],
        out_hbm.at[pl.ds(c * NPAD + s * RPT, RPT)],
    )


NCH = EPC // K
IB = 20
NBLKS = NCH // IB


@functools.partial(
    pl.kernel,
    out_type=jax.ShapeDtypeStruct((2 * NPAD, CH), jnp.float32),
    mesh=_mesh,
    scratch_types=[
        pltpu.VMEM((IB, 2, K), jnp.int32),
        pltpu.VMEM((IB, 2, K), jnp.int32),
        pltpu.VMEM((K, CH), jnp.float32),
        pltpu.VMEM((K, CH), jnp.float32),
        pltpu.VMEM_SHARED((NPAD, CH), jnp.float32),
        pltpu.SemaphoreType.DMA,
        pltpu.SemaphoreType.DMA,
        pltpu.SemaphoreType.DMA,
        pltpu.SemaphoreType.DMA,
    ],
)
def _agg_kernel(s_hbm, sd4_hbm, zz_hbm, out_hbm,
                sd0, sd1, rows0, rows1, acc_sh, semx0, semx1, semg0, semg1):
    c = lax.axis_index("c")
    s = lax.axis_index("s")
    w = c * 16 + s
    pltpu.async_copy(sd4_hbm.at[w, pl.ds(0, IB)], sd0, semx0)
    pltpu.sync_copy(
        zz_hbm.at[pl.ds(s * RPT, RPT)], acc_sh.at[pl.ds(s * RPT, RPT)]
    )
    plsc.subcore_barrier()

    for b in range(NBLKS):
        sd_c, semx_c = (sd0, semx0) if b % 2 == 0 else (sd1, semx1)
        sd_n, semx_n = (sd1, semx1) if b % 2 == 0 else (sd0, semx0)
        if b + 1 < NBLKS:
            pltpu.async_copy(
                sd4_hbm.at[w, pl.ds((b + 1) * IB, IB)], sd_n, semx_n
            )
        pltpu.make_async_copy(
            sd4_hbm.at[w, pl.ds(b * IB, IB)], sd_c, semx_c
        ).wait()

        pltpu.async_copy(s_hbm.at[sd_c.at[0, 0]], rows0, semg0)

        def body(i, carry, sd_c=sd_c):
            j0 = i * 2
            j1 = j0 + 1
            pltpu.async_copy(s_hbm.at[sd_c.at[j1, 0]], rows1, semg1)
            pltpu.make_async_copy(s_hbm.at[sd_c.at[j0, 0]], rows0, semg0).wait()
            pltpu.sync_copy(rows0, acc_sh.at[sd_c.at[j0, 1]], add=True)

            @pl.when(i < IB // 2 - 1)
            def _():
                pltpu.async_copy(s_hbm.at[sd_c.at[j0 + 2, 0]], rows0, semg0)

            pltpu.make_async_copy(s_hbm.at[sd_c.at[j1, 0]], rows1, semg1).wait()
            pltpu.sync_copy(rows1, acc_sh.at[sd_c.at[j1, 1]], add=True)
            return carry

        lax.fori_loop(0, IB // 2, body, 0)

    plsc.subcore_barrier()
    pltpu.sync_copy(
        acc_sh.at[pl.ds(s * RPT, RPT)],
        out_hbm.at[pl.ds(c * NPAD + s * RPT, RPT)],
    )


def _b1_body(x_ref, w_ref, g0_ref, g1_ref, s_ref, d_ref):
    d = lax.rsqrt(g0_ref[0, 0] + g1_ref[0, 0] + 1.0)
    h = jnp.dot(x_ref[...], w_ref[...], preferred_element_type=jnp.float32)
    sv = h * d[:, None]
    s_ref[0] = sv[:, :CH]
    s_ref[1] = sv[:, CH:]
    d_ref[0, 0] = d


def _b2_body(a_ref, s_ref, d_ref, b_ref, w_ref, o_ref):
    i = pl.program_id(0)
    a = jnp.concatenate([a_ref[0], a_ref[1]], axis=1)
    sf = jnp.concatenate([s_ref[0], s_ref[1]], axis=1)
    d = d_ref[0, 0]
    h = jnp.maximum((a + sf) * d[:, None] + b_ref[0][None, :], 0.0)
    s2 = jnp.dot(h, w_ref[...], preferred_element_type=jnp.float32)
    s2 = s2 * d[:, None]
    rid = i * RB + lax.broadcasted_iota(jnp.int32, (RB, 1), 0)
    s2 = jnp.where(rid < N, s2, 0.0)
    o_ref[0] = s2[:, :CH]
    o_ref[1] = s2[:, CH:]


def _b3_body(a_ref, s_ref, d_ref, b_ref, bat_ref, lw_ref, os_ref, oc_ref):
    i = pl.program_id(0)
    a = jnp.concatenate([a_ref[0], a_ref[1]], axis=1)
    sf = jnp.concatenate([s_ref[0], s_ref[1]], axis=1)
    d = d_ref[0, 0]
    h = jnp.maximum((a + sf) * d[:, None] + b_ref[0][None, :], 0.0)
    y = jnp.dot(h, lw_ref[...], preferred_element_type=jnp.float32)
    bat = bat_ref[0, 0]
    gid = lax.broadcasted_iota(jnp.int32, (NG, 1), 0)
    m = (bat[None, :] == gid).astype(jnp.float32)
    ps = jnp.dot(m, y, preferred_element_type=jnp.float32)
    pc = jnp.broadcast_to(jnp.sum(m, axis=1, keepdims=True), (NG, 128))

    @pl.when(i == 0)
    def _():
        os_ref[...] = jnp.zeros_like(os_ref)
        oc_ref[...] = jnp.zeros_like(oc_ref)

    os_ref[...] += ps
    oc_ref[...] += pc


def _b1_call(xp, W1, deg0, deg1):
    return pl.pallas_call(
        _b1_body,
        grid=(NBLK,),
        in_specs=[
            pl.BlockSpec((RB, INC), lambda i: (i, 0)),
            pl.BlockSpec((INC, HID), lambda i: (0, 0)),
            pl.BlockSpec((1, 1, RB), lambda i: (i, 0, 0)),
            pl.BlockSpec((1, 1, RB), lambda i: (i, 0, 0)),
        ],
        out_specs=[
            pl.BlockSpec((2, RB, CH), lambda i: (0, i, 0)),
            pl.BlockSpec((1, 1, RB), lambda i: (i, 0, 0)),
        ],
        out_shape=[
            jax.ShapeDtypeStruct((2, NPAD, CH), jnp.float32),
            jax.ShapeDtypeStruct((NBLK, 1, RB), jnp.float32),
        ],
    )(xp, W1, deg0, deg1)


def _b2_call(agg1, s1, d, b1r, W2):
    return pl.pallas_call(
        _b2_body,
        grid=(NBLK,),
        in_specs=[
            pl.BlockSpec((2, RB, CH), lambda i: (0, i, 0)),
            pl.BlockSpec((2, RB, CH), lambda i: (0, i, 0)),
            pl.BlockSpec((1, 1, RB), lambda i: (i, 0, 0)),
            pl.BlockSpec((1, HID), lambda i: (0, 0)),
            pl.BlockSpec((HID, HID), lambda i: (0, 0)),
        ],
        out_specs=pl.BlockSpec((2, RB, CH), lambda i: (0, i, 0)),
        out_shape=jax.ShapeDtypeStruct((2, NPAD, CH), jnp.float32),
    )(agg1, s1, d, b1r, W2)


def _b3_call(agg2, s2, d, b2r, batp, lwp):
    return pl.pallas_call(
        _b3_body,
        grid=(NBLK,),
        in_specs=[
            pl.BlockSpec((2, RB, CH), lambda i: (0, i, 0)),
            pl.BlockSpec((2, RB, CH), lambda i: (0, i, 0)),
            pl.BlockSpec((1, 1, RB), lambda i: (i, 0, 0)),
            pl.BlockSpec((1, HID), lambda i: (0, 0)),
            pl.BlockSpec((1, 1, RB), lambda i: (i, 0, 0)),
            pl.BlockSpec((HID, 128), lambda i: (0, 0)),
        ],
        out_specs=[
            pl.BlockSpec((NG, 128), lambda i: (0, 0)),
            pl.BlockSpec((NG, 128), lambda i: (0, 0)),
        ],
        out_shape=[
            jax.ShapeDtypeStruct((NG, 128), jnp.float32),
            jax.ShapeDtypeStruct((NG, 128), jnp.float32),
        ],
    )(agg2, s2, d, b2r, batp, lwp)


def kernel(x, edge_index, batch, W1, b1, W2, b2, lin_W, lin_b):
    src = edge_index[0].astype(jnp.int32)
    dst = edge_index[1].astype(jnp.int32)
    pad = jnp.full((EPAD - E,), N, jnp.int32)
    srcp = jnp.concatenate([src, pad])
    dstp = jnp.concatenate([dst, pad])
    src_r = srcp.reshape(16, EPC // K, 1, K)
    dst_r = dstp.reshape(16, EPC // K, 1, K)
    sd4 = jnp.concatenate(
        [
            jnp.concatenate([src_r, dst_r], axis=2),
            jnp.concatenate([src_r + NPAD, dst_r], axis=2),
        ],
        axis=0,
    )
    xp = jnp.zeros((NPAD, INC), jnp.float32).at[:N].set(x)
    zz = jnp.zeros((NPAD, CH), jnp.float32)
    batp = jnp.concatenate(
        [batch.astype(jnp.int32), jnp.full((NPAD - N,), NG, jnp.int32)]
    ).reshape(NBLK, 1, RB)
    b1r = b1.reshape(1, HID)
    b2r = b2.reshape(1, HID)
    lwp = jnp.pad(lin_W, ((0, 0), (0, 128 - lin_W.shape[1])))

    ones1 = jnp.zeros((K, CH), jnp.float32).at[:, 0].set(1.0)
    dst4 = dstp.reshape(32, EPD // K, K)
    degout = _deg_kernel(dst4, ones1, zz)
    deg0 = degout[:NPAD, 0].reshape(NBLK, 1, RB)
    deg1 = degout[NPAD:, 0].reshape(NBLK, 1, RB)

    s1, d = _b1_call(xp, W1, deg0, deg1)
    agg1 = _agg_kernel(s1.reshape(2 * NPAD, CH), sd4, zz)
    s2 = _b2_call(agg1.reshape(2, NPAD, CH), s1, d, b1r, W2)
    agg2 = _agg_kernel(s2.reshape(2 * NPAD, CH), sd4, zz)
    osum, ocnt = _b3_call(agg2.reshape(2, NPAD, CH), s2, d, b2r, batp, lwp)

    return osum[:, 0] / jnp.maximum(ocnt[:, 0], 1.0) + lin_b[0]


# speedup vs baseline: 1.0630x; 1.0001x over previous
import functools

import jax
import jax.numpy as jnp
from jax import lax
from jax.experimental import pallas as pl
from jax.experimental.pallas import tpu as pltpu
from jax.experimental.pallas import tpu_sc as plsc

N = 10000
NPAD = 10240
E = 320000
EPAD = 327680
INC = 128
HID = 256
CH = 128
NG = 64
RB = 256
NBLK = NPAD // RB
K = 128
EPC = EPAD // 16
EPD = EPAD // 32
RPT = NPAD // 16

_mesh = plsc.VectorSubcoreMesh(core_axis_name="c", subcore_axis_name="s")


NDCH = EPD // K
DG = 8


@functools.partial(
    pl.kernel,
    out_type=jax.ShapeDtypeStruct((2 * NPAD, CH), jnp.float32),
    mesh=_mesh,
    scratch_types=[
        pltpu.VMEM((NDCH, K), jnp.int32),
        pltpu.VMEM((K, CH), jnp.float32),
        pltpu.VMEM_SHARED((NPAD, CH), jnp.float32),
        pltpu.SemaphoreType.DMA,
        pltpu.SemaphoreType.DMA,
    ],
)
def _deg_kernel(dst3_hbm, ones_hbm, zz_hbm, out_hbm,
                dsts_v, ones_v, deg_sh, semA, semB):
    c = lax.axis_index("c")
    s = lax.axis_index("s")
    wid = s * 2 + c
    pltpu.sync_copy(ones_hbm, ones_v)
    pltpu.sync_copy(dst3_hbm.at[wid], dsts_v)
    pltpu.sync_copy(
        zz_hbm.at[pl.ds(s * RPT, RPT)], deg_sh.at[pl.ds(s * RPT, RPT)]
    )
    plsc.subcore_barrier()

    def fire(g, sem):
        for t in range(DG):
            pltpu.async_copy(
                ones_v, deg_sh.at[dsts_v.at[g * DG + t]], sem, add=True
            )

    def drain(g, sem):
        for t in range(DG):
            pltpu.make_async_copy(
                ones_v, deg_sh.at[dsts_v.at[g * DG + t]], sem
            ).wait()

    NGRP = NDCH // DG
    fire(0, semA)
    for g in range(NGRP):
        cur = semA if g % 2 == 0 else semB
        nxt = semB if g % 2 == 0 else semA
        if g + 1 < NGRP:
            fire(g + 1, nxt)
        drain(g, cur)

    plsc.subcore_barrier()
    pltpu.sync_copy(
        deg_sh.at[pl.ds(s * RPT, RPT)],
        out_hbm.at[pl.ds(c * NPAD + s * RPT, RPT)],
    )


NCH = EPC // K
IB = 20
NBLKS = NCH // IB


@functools.partial(
    pl.kernel,
    out_type=jax.ShapeDtypeStruct((2 * NPAD, CH), jnp.float32),
    mesh=_mesh,
    scratch_types=[
        pltpu.VMEM((IB, 2, K), jnp.int32),
        pltpu.VMEM((IB, 2, K), jnp.int32),
        pltpu.VMEM((K, CH), jnp.float32),
        pltpu.VMEM((K, CH), jnp.float32),
        pltpu.VMEM_SHARED((NPAD, CH), jnp.float32),
        pltpu.SemaphoreType.DMA,
        pltpu.SemaphoreType.DMA,
        pltpu.SemaphoreType.DMA,
        pltpu.SemaphoreType.DMA,
    ],
)
def _agg_kernel(s_hbm, sd4_hbm, zz_hbm, out_hbm,
                sd0, sd1, rows0, rows1, acc_sh, semx0, semx1, semg0, semg1):
    c = lax.axis_index("c")
    s = lax.axis_index("s")
    w = c * 16 + s
    pltpu.async_copy(sd4_hbm.at[w, pl.ds(0, IB)], sd0, semx0)
    pltpu.sync_copy(
        zz_hbm.at[pl.ds(s * RPT, RPT)], acc_sh.at[pl.ds(s * RPT, RPT)]
    )
    plsc.subcore_barrier()

    for b in range(NBLKS):
        sd_c, semx_c = (sd0, semx0) if b % 2 == 0 else (sd1, semx1)
        sd_n, semx_n = (sd1, semx1) if b % 2 == 0 else (sd0, semx0)
        if b + 1 < NBLKS:
            pltpu.async_copy(
                sd4_hbm.at[w, pl.ds((b + 1) * IB, IB)], sd_n, semx_n
            )
        pltpu.make_async_copy(
            sd4_hbm.at[w, pl.ds(b * IB, IB)], sd_c, semx_c
        ).wait()

        pltpu.async_copy(s_hbm.at[sd_c.at[0, 0]], rows0, semg0)

        def body(i, carry, sd_c=sd_c):
            j0 = i * 2
            j1 = j0 + 1
            pltpu.async_copy(s_hbm.at[sd_c.at[j1, 0]], rows1, semg1)
            pltpu.make_async_copy(s_hbm.at[sd_c.at[j0, 0]], rows0, semg0).wait()
            pltpu.sync_copy(rows0, acc_sh.at[sd_c.at[j0, 1]], add=True)

            @pl.when(i < IB // 2 - 1)
            def _():
                pltpu.async_copy(s_hbm.at[sd_c.at[j0 + 2, 0]], rows0, semg0)

            pltpu.make_async_copy(s_hbm.at[sd_c.at[j1, 0]], rows1, semg1).wait()
            pltpu.sync_copy(rows1, acc_sh.at[sd_c.at[j1, 1]], add=True)
            return carry

        lax.fori_loop(0, IB // 2, body, 0)

    plsc.subcore_barrier()
    pltpu.sync_copy(
        acc_sh.at[pl.ds(s * RPT, RPT)],
        out_hbm.at[pl.ds(c * NPAD + s * RPT, RPT)],
    )


def _b1_body(x_ref, w_ref, g0_ref, g1_ref, s_ref, d_ref):
    d = lax.rsqrt(g0_ref[0, 0] + g1_ref[0, 0] + 1.0)
    h = jnp.dot(x_ref[...], w_ref[...], preferred_element_type=jnp.float32)
    sv = h * d[:, None]
    s_ref[0] = sv[:, :CH]
    s_ref[1] = sv[:, CH:]
    d_ref[0, 0] = d


def _b2_body(a_ref, s_ref, d_ref, b_ref, w_ref, o_ref):
    i = pl.program_id(0)
    a = jnp.concatenate([a_ref[0], a_ref[1]], axis=1)
    sf = jnp.concatenate([s_ref[0], s_ref[1]], axis=1)
    d = d_ref[0, 0]
    h = jnp.maximum((a + sf) * d[:, None] + b_ref[0][None, :], 0.0)
    s2 = jnp.dot(h, w_ref[...], preferred_element_type=jnp.float32)
    s2 = s2 * d[:, None]
    rid = i * RB + lax.broadcasted_iota(jnp.int32, (RB, 1), 0)
    s2 = jnp.where(rid < N, s2, 0.0)
    o_ref[0] = s2[:, :CH]
    o_ref[1] = s2[:, CH:]


def _b3_body(a_ref, s_ref, d_ref, b_ref, bat_ref, lw_ref, os_ref, oc_ref):
    i = pl.program_id(0)
    a = jnp.concatenate([a_ref[0], a_ref[1]], axis=1)
    sf = jnp.concatenate([s_ref[0], s_ref[1]], axis=1)
    d = d_ref[0, 0]
    h = jnp.maximum((a + sf) * d[:, None] + b_ref[0][None, :], 0.0)
    y = jnp.dot(h, lw_ref[...], preferred_element_type=jnp.float32)
    bat = bat_ref[0, 0]
    gid = lax.broadcasted_iota(jnp.int32, (NG, 1), 0)
    m = (bat[None, :] == gid).astype(jnp.float32)
    ps = jnp.dot(m, y, preferred_element_type=jnp.float32)
    pc = jnp.broadcast_to(jnp.sum(m, axis=1, keepdims=True), (NG, 128))

    @pl.when(i == 0)
    def _():
        os_ref[...] = jnp.zeros_like(os_ref)
        oc_ref[...] = jnp.zeros_like(oc_ref)

    os_ref[...] += ps
    oc_ref[...] += pc


def _b1_call(xp, W1, deg0, deg1):
    return pl.pallas_call(
        _b1_body,
        grid=(NBLK,),
        in_specs=[
            pl.BlockSpec((RB, INC), lambda i: (i, 0)),
            pl.BlockSpec((INC, HID), lambda i: (0, 0)),
            pl.BlockSpec((1, 1, RB), lambda i: (i, 0, 0)),
            pl.BlockSpec((1, 1, RB), lambda i: (i, 0, 0)),
        ],
        out_specs=[
            pl.BlockSpec((2, RB, CH), lambda i: (0, i, 0)),
            pl.BlockSpec((1, 1, RB), lambda i: (i, 0, 0)),
        ],
        out_shape=[
            jax.ShapeDtypeStruct((2, NPAD, CH), jnp.float32),
            jax.ShapeDtypeStruct((NBLK, 1, RB), jnp.float32),
        ],
    )(xp, W1, deg0, deg1)


def _b2_call(agg1, s1, d, b1r, W2):
    return pl.pallas_call(
        _b2_body,
        grid=(NBLK,),
        in_specs=[
            pl.BlockSpec((2, RB, CH), lambda i: (0, i, 0)),
            pl.BlockSpec((2, RB, CH), lambda i: (0, i, 0)),
            pl.BlockSpec((1, 1, RB), lambda i: (i, 0, 0)),
            pl.BlockSpec((1, HID), lambda i: (0, 0)),
            pl.BlockSpec((HID, HID), lambda i: (0, 0)),
        ],
        out_specs=pl.BlockSpec((2, RB, CH), lambda i: (0, i, 0)),
        out_shape=jax.ShapeDtypeStruct((2, NPAD, CH), jnp.float32),
    )(agg1, s1, d, b1r, W2)


def _b3_call(agg2, s2, d, b2r, batp, lwp):
    return pl.pallas_call(
        _b3_body,
        grid=(NBLK,),
        in_specs=[
            pl.BlockSpec((2, RB, CH), lambda i: (0, i, 0)),
            pl.BlockSpec((2, RB, CH), lambda i: (0, i, 0)),
            pl.BlockSpec((1, 1, RB), lambda i: (i, 0, 0)),
            pl.BlockSpec((1, HID), lambda i: (0, 0)),
            pl.BlockSpec((1, 1, RB), lambda i: (i, 0, 0)),
            pl.BlockSpec((HID, 128), lambda i: (0, 0)),
        ],
        out_specs=[
            pl.BlockSpec((NG, 128), lambda i: (0, 0)),
            pl.BlockSpec((NG, 128), lambda i: (0, 0)),
        ],
        out_shape=[
            jax.ShapeDtypeStruct((NG, 128), jnp.float32),
            jax.ShapeDtypeStruct((NG, 128), jnp.float32),
        ],
    )(agg2, s2, d, b2r, batp, lwp)


def kernel(x, edge_index, batch, W1, b1, W2, b2, lin_W, lin_b):
    src = edge_index[0].astype(jnp.int32)
    dst = edge_index[1].astype(jnp.int32)
    pad = jnp.full((EPAD - E,), N, jnp.int32)
    srcp = jnp.concatenate([src, pad])
    dstp = jnp.concatenate([dst, pad])
    src_r = srcp.reshape(16, EPC // K, 1, K)
    dst_r = dstp.reshape(16, EPC // K, 1, K)
    sd4 = jnp.concatenate(
        [
            jnp.concatenate([src_r, dst_r], axis=2),
            jnp.concatenate([src_r + NPAD, dst_r], axis=2),
        ],
        axis=0,
    )
    xp = jnp.zeros((NPAD, INC), jnp.float32).at[:N].set(x)
    zz = jnp.zeros((NPAD, CH), jnp.float32)
    batp = jnp.concatenate(
        [batch.astype(jnp.int32), jnp.full((NPAD - N,), NG, jnp.int32)]
    ).reshape(NBLK, 1, RB)
    b1r = b1.reshape(1, HID)
    b2r = b2.reshape(1, HID)
    lwp = jnp.pad(lin_W, ((0, 0), (0, 128 - lin_W.shape[1])))

    ones1 = jnp.zeros((K, CH), jnp.float32).at[:, 0].set(1.0)
    dst4 = dstp.reshape(32, EPD // K, K)
    degout = _deg_kernel(dst4, ones1, zz)
    deg0 = degout[:NPAD, 0].reshape(NBLK, 1, RB)
    deg1 = degout[NPAD:, 0].reshape(NBLK, 1, RB)

    s1, d = _b1_call(xp, W1, deg0, deg1)
    agg1 = _agg_kernel(s1.reshape(2 * NPAD, CH), sd4, zz)
    s2 = _b2_call(agg1.reshape(2, NPAD, CH), s1, d, b1r, W2)
    agg2 = _agg_kernel(s2.reshape(2 * NPAD, CH), sd4, zz)
    osum, ocnt = _b3_call(agg2.reshape(2, NPAD, CH), s2, d, b2r, batp, lwp)

    return osum[:, 0] / jnp.maximum(ocnt[:, 0], 1.0) + lin_b[0]
